# baseline (speedup 1.0000x reference)
import functools

import jax
import jax.numpy as jnp
from jax import lax
from jax.experimental import pallas as pl
from jax.experimental.pallas import tpu as pltpu
from jax.experimental.pallas import tpu_sc as plsc

T = 2048
D = 768
E = 8
B = 128
NCHUNK = T // B
NSLOT = T + E * B
NBLK = NSLOT // B
NBLK_PAD = 32

NC = 2
NS = 16
NW = NC * NS
TPW = T // NW


def _router_body(x_ref, wr_ref, br_ref, oh_ref, dest_ref, bexp_ref):
    f32 = jnp.float32
    logits = jnp.dot(x_ref[...], wr_ref[...], preferred_element_type=f32)
    logits = logits + br_ref[...]
    m = jnp.max(logits, axis=1, keepdims=True)
    col = lax.broadcasted_iota(jnp.int32, (T, E), 1).astype(f32)
    first = jnp.min(jnp.where(logits == m, col, 1e9), axis=1, keepdims=True)
    oh = (col == first).astype(f32)
    oh_ref[...] = oh

    seg = (lax.broadcasted_iota(jnp.int32, (NCHUNK, T), 0)
           == lax.shift_right_logical(
               lax.broadcasted_iota(jnp.int32, (NCHUNK, T), 1), 7)
           ).astype(f32)
    chunk_tot = jnp.dot(seg, oh, preferred_element_type=f32)
    ltx = (lax.broadcasted_iota(jnp.int32, (NCHUNK, NCHUNK), 0)
           > lax.broadcasted_iota(jnp.int32, (NCHUNK, NCHUNK), 1)).astype(f32)
    chunk_excl = jnp.dot(ltx, chunk_tot, preferred_element_type=f32)
    lt = (lax.broadcasted_iota(jnp.int32, (B, B), 0)
          >= lax.broadcasted_iota(jnp.int32, (B, B), 1)).astype(f32)
    parts = []
    for c in range(NCHUNK):
        blk = oh[c * B:(c + 1) * B, :]
        s = jnp.dot(lt, blk, preferred_element_type=f32)
        parts.append(s + chunk_excl[c:c + 1, :])
    rank_incl = jnp.concatenate(parts, axis=0)
    counts = chunk_excl[NCHUNK - 1:NCHUNK, :] + chunk_tot[NCHUNK - 1:NCHUNK, :]

    padded = jnp.floor((counts + (B - 1)) * (1.0 / B)) * B
    ut = (lax.broadcasted_iota(jnp.int32, (E, E), 0)
          <= lax.broadcasted_iota(jnp.int32, (E, E), 1)).astype(f32)
    offs_excl = jnp.dot(padded, ut, preferred_element_type=f32) - padded

    dest = jnp.sum(oh * (offs_excl + rank_incl - 1.0), axis=1, keepdims=True)
    dest_ref[...] = dest.astype(jnp.int32)

    ib = lax.broadcasted_iota(jnp.int32, (NBLK_PAD, E), 0).astype(f32) * B
    act = jnp.logical_and(offs_excl <= ib, ib < offs_excl + padded)
    lane = lax.broadcasted_iota(jnp.int32, (NBLK_PAD, E), 1).astype(f32)
    actf = act.astype(f32)
    bexp = (jnp.sum(actf * lane, axis=1, keepdims=True)
            + jnp.sum(actf, axis=1, keepdims=True) - 1.0)
    bexp_ref[...] = bexp.astype(jnp.int32)


def _ffn_body(blk_ref, xs_ref, we_ref, be_ref, wo_ref, bo_ref, out_ref):
    i = pl.program_id(0)
    e = blk_ref[i]

    @pl.when(e >= 0)
    def _():
        f32 = jnp.float32
        bf16 = jnp.bfloat16
        w = we_ref[pl.ds(e, 1)].reshape(D, D)
        h = jnp.dot(xs_ref[...].astype(bf16), w, preferred_element_type=f32)
        h = h + be_ref[pl.ds(e, 1), :]
        out_ref[...] = (jnp.dot(h.astype(bf16), wo_ref[...],
                                preferred_element_type=f32)
                        + bo_ref[...])


def _sc_scatter_body(x_hbm, dest_hbm, out_hbm, idx_v, rows_v, sem):
    wid = lax.axis_index("s") * NC + lax.axis_index("c")
    base = wid * TPW
    pltpu.sync_copy(dest_hbm.at[pl.ds(base, TPW)], idx_v)
    pltpu.sync_copy(x_hbm.at[pl.ds(base, TPW)], rows_v)
    pltpu.async_copy(rows_v, out_hbm.at[idx_v], sem).wait()


def _sc_gather_body(slots_hbm, dest_hbm, out_hbm, idx_v, rows_v, sem):
    wid = lax.axis_index("s") * NC + lax.axis_index("c")
    base = wid * TPW
    pltpu.sync_copy(dest_hbm.at[pl.ds(base, TPW)], idx_v)
    pltpu.async_copy(slots_hbm.at[idx_v], rows_v, sem).wait()
    pltpu.sync_copy(rows_v, out_hbm.at[pl.ds(base, TPW)])


@functools.cache
def _sc_mesh():
    return plsc.VectorSubcoreMesh(
        core_axis_name="c", subcore_axis_name="s",
        num_cores=NC, num_subcores=NS)


def _router_call(x, Wr, br2):
    return pl.pallas_call(
        _router_body,
        out_shape=(
            jax.ShapeDtypeStruct((T, E), jnp.float32),
            jax.ShapeDtypeStruct((T, 1), jnp.int32),
            jax.ShapeDtypeStruct((NBLK_PAD, 1), jnp.int32),
        ),
    )(x, Wr, br2)


def _scatter_call(x, dest):
    fn = pl.kernel(
        _sc_scatter_body,
        out_type=jax.ShapeDtypeStruct((NSLOT, D), jnp.float32),
        mesh=_sc_mesh(),
        scratch_types=[
            pltpu.VMEM((TPW,), jnp.int32),
            pltpu.VMEM((TPW, D), jnp.float32),
            pltpu.SemaphoreType.DMA,
        ],
    )
    return fn(x, dest)


def _ffn_call(bexp, xs, We16, be, Wo16, bo2):
    return pl.pallas_call(
        _ffn_body,
        grid=(NBLK,),
        in_specs=[
            pl.BlockSpec(memory_space=pltpu.SMEM),
            pl.BlockSpec((B, D), lambda i: (i, 0)),
            pl.BlockSpec((E, D, D), lambda i: (0, 0, 0)),
            pl.BlockSpec((E, D), lambda i: (0, 0)),
            pl.BlockSpec((D, D), lambda i: (0, 0)),
            pl.BlockSpec((1, D), lambda i: (0, 0)),
        ],
        out_specs=pl.BlockSpec((B, D), lambda i: (i, 0)),
        out_shape=jax.ShapeDtypeStruct((NSLOT, D), jnp.float32),
    )(bexp, xs, We16, be, Wo16, bo2)


def _gather_call(slots, dest):
    fn = pl.kernel(
        _sc_gather_body,
        out_type=jax.ShapeDtypeStruct((T, D), jnp.float32),
        mesh=_sc_mesh(),
        scratch_types=[
            pltpu.VMEM((TPW,), jnp.int32),
            pltpu.VMEM((TPW, D), jnp.float32),
            pltpu.SemaphoreType.DMA,
        ],
    )
    return fn(slots, dest)


def kernel(x, Wr, br, We, be, Wo, bo):
    oh, dest2d, bexp2d = _router_call(x, Wr, br.reshape(1, E))
    dest = dest2d.reshape(T)
    bexp = bexp2d.reshape(NBLK_PAD)
    xs = _scatter_call(x, dest)
    slots = _ffn_call(bexp, xs, We.astype(jnp.bfloat16), be,
                      Wo.astype(jnp.bfloat16), bo.reshape(1, D))
    output = _gather_call(slots, dest)
    return (output, oh)


# speedup vs baseline: 1.0937x; 1.0937x over previous
import functools

import jax
import jax.numpy as jnp
from jax import lax
from jax.experimental import pallas as pl
from jax.experimental.pallas import tpu as pltpu
from jax.experimental.pallas import tpu_sc as plsc

T = 2048
D = 768
E = 8
B = 128
NCHUNK = T // B
NSLOT = T + E * B
NBLK = NSLOT // B
NBLK_PAD = 32

NC = 2
NS = 16
NW = NC * NS
TPW = T // NW


def _router_body(x_ref, wr_ref, br_ref, oh_ref, dest_ref, bexp_ref):
    f32 = jnp.float32
    logits = jnp.dot(x_ref[...], wr_ref[...], preferred_element_type=f32)
    logits = logits + br_ref[...]
    m = jnp.max(logits, axis=1, keepdims=True)
    col = lax.broadcasted_iota(jnp.int32, (T, E), 1).astype(f32)
    first = jnp.min(jnp.where(logits == m, col, 1e9), axis=1, keepdims=True)
    oh = (col == first).astype(f32)
    oh_ref[...] = oh

    seg = (lax.broadcasted_iota(jnp.int32, (NCHUNK, T), 0)
           == lax.shift_right_logical(
               lax.broadcasted_iota(jnp.int32, (NCHUNK, T), 1), 7)
           ).astype(f32)
    chunk_tot = jnp.dot(seg, oh, preferred_element_type=f32)
    ltx = (lax.broadcasted_iota(jnp.int32, (NCHUNK, NCHUNK), 0)
           > lax.broadcasted_iota(jnp.int32, (NCHUNK, NCHUNK), 1)).astype(f32)
    chunk_excl = jnp.dot(ltx, chunk_tot, preferred_element_type=f32)
    lt = (lax.broadcasted_iota(jnp.int32, (B, B), 0)
          >= lax.broadcasted_iota(jnp.int32, (B, B), 1)).astype(f32)
    parts = []
    for c in range(NCHUNK):
        blk = oh[c * B:(c + 1) * B, :]
        s = jnp.dot(lt, blk, preferred_element_type=f32)
        parts.append(s + chunk_excl[c:c + 1, :])
    rank_incl = jnp.concatenate(parts, axis=0)
    counts = chunk_excl[NCHUNK - 1:NCHUNK, :] + chunk_tot[NCHUNK - 1:NCHUNK, :]

    padded = jnp.floor((counts + (B - 1)) * (1.0 / B)) * B
    ut = (lax.broadcasted_iota(jnp.int32, (E, E), 0)
          <= lax.broadcasted_iota(jnp.int32, (E, E), 1)).astype(f32)
    offs_excl = jnp.dot(padded, ut, preferred_element_type=f32) - padded

    dest = jnp.sum(oh * (offs_excl + rank_incl - 1.0), axis=1, keepdims=True)
    dest_ref[...] = dest.astype(jnp.int32)

    ib = lax.broadcasted_iota(jnp.int32, (NBLK_PAD, E), 0).astype(f32) * B
    act = jnp.logical_and(offs_excl <= ib, ib < offs_excl + padded)
    lane = lax.broadcasted_iota(jnp.int32, (NBLK_PAD, E), 1).astype(f32)
    actf = act.astype(f32)
    bexp = (jnp.sum(actf * lane, axis=1, keepdims=True)
            + jnp.sum(actf, axis=1, keepdims=True) - 1.0)
    bexp_ref[...] = bexp.astype(jnp.int32)


def _ffn_body(blk_ref, xs_ref, we_ref, be_ref, wo_ref, bo_ref, out_ref):
    i = pl.program_id(0)
    e = blk_ref[i]

    @pl.when(e >= 0)
    def _():
        f32 = jnp.float32
        w = we_ref[pl.ds(e, 1)].reshape(D, D)
        h = jnp.dot(xs_ref[...], w, preferred_element_type=f32)
        h = h + be_ref[pl.ds(e, 1), :]
        out_ref[...] = (jnp.dot(h, wo_ref[...], preferred_element_type=f32)
                        + bo_ref[...])


def _sc_scatter_body(x_hbm, dest_hbm, out_hbm, idx_v, rows_v, sem):
    wid = lax.axis_index("s") * NC + lax.axis_index("c")
    base = wid * TPW
    pltpu.sync_copy(dest_hbm.at[pl.ds(base, TPW)], idx_v)
    pltpu.sync_copy(x_hbm.at[pl.ds(base, TPW)], rows_v)
    pltpu.async_copy(rows_v, out_hbm.at[idx_v], sem).wait()


def _sc_gather_body(slots_hbm, dest_hbm, out_hbm, idx_v, rows_v, sem):
    wid = lax.axis_index("s") * NC + lax.axis_index("c")
    base = wid * TPW
    pltpu.sync_copy(dest_hbm.at[pl.ds(base, TPW)], idx_v)
    pltpu.async_copy(slots_hbm.at[idx_v], rows_v, sem).wait()
    pltpu.sync_copy(rows_v, out_hbm.at[pl.ds(base, TPW)])


@functools.cache
def _sc_mesh():
    return plsc.VectorSubcoreMesh(
        core_axis_name="c", subcore_axis_name="s",
        num_cores=NC, num_subcores=NS)


def _router_call(x, Wr, br2):
    return pl.pallas_call(
        _router_body,
        out_shape=(
            jax.ShapeDtypeStruct((T, E), jnp.float32),
            jax.ShapeDtypeStruct((T, 1), jnp.int32),
            jax.ShapeDtypeStruct((NBLK_PAD, 1), jnp.int32),
        ),
    )(x, Wr, br2)


def _scatter_call(x, dest):
    fn = pl.kernel(
        _sc_scatter_body,
        out_type=jax.ShapeDtypeStruct((NSLOT, D), jnp.float32),
        mesh=_sc_mesh(),
        scratch_types=[
            pltpu.VMEM((TPW,), jnp.int32),
            pltpu.VMEM((TPW, D), jnp.float32),
            pltpu.SemaphoreType.DMA,
        ],
    )
    return fn(x, dest)


def _ffn_call(bexp, xs, We16, be, Wo16, bo2):
    return pl.pallas_call(
        _ffn_body,
        grid=(NBLK,),
        in_specs=[
            pl.BlockSpec(memory_space=pltpu.SMEM),
            pl.BlockSpec((B, D), lambda i: (i, 0)),
            pl.BlockSpec((E, D, D), lambda i: (0, 0, 0)),
            pl.BlockSpec((E, D), lambda i: (0, 0)),
            pl.BlockSpec((D, D), lambda i: (0, 0)),
            pl.BlockSpec((1, D), lambda i: (0, 0)),
        ],
        out_specs=pl.BlockSpec((B, D), lambda i: (i, 0)),
        out_shape=jax.ShapeDtypeStruct((NSLOT, D), jnp.float32),
    )(bexp, xs, We16, be, Wo16, bo2)


def _gather_call(slots, dest):
    fn = pl.kernel(
        _sc_gather_body,
        out_type=jax.ShapeDtypeStruct((T, D), jnp.float32),
        mesh=_sc_mesh(),
        scratch_types=[
            pltpu.VMEM((TPW,), jnp.int32),
            pltpu.VMEM((TPW, D), jnp.float32),
            pltpu.SemaphoreType.DMA,
        ],
    )
    return fn(slots, dest)


def kernel(x, Wr, br, We, be, Wo, bo):
    oh, dest2d, bexp2d = _router_call(x, Wr, br.reshape(1, E))
    dest = dest2d.reshape(T)
    bexp = bexp2d.reshape(NBLK_PAD)
    xs = _scatter_call(x, dest)
    slots = _ffn_call(bexp, xs, We, be, Wo, bo.reshape(1, D))
    output = _gather_call(slots, dest)
    return (output, oh)
